# baseline (speedup 1.0000x reference)
import functools

import jax
import jax.numpy as jnp
from jax import lax
from jax.experimental import pallas as pl
from jax.experimental.pallas import tpu as pltpu
from jax.experimental.pallas import tpu_sc as plsc

D_MODEL = 1024
D_FF = 2816
N_EXP = 8
EPAD = 128
T = 2048
A = 4096
BM = 512
NB = 15
PMAX = NB * BM
NW = 32
L = 16

_sc_mesh = functools.partial(
    plsc.VectorSubcoreMesh, core_axis_name="c", subcore_axis_name="s"
)


def _wid():
    return lax.axis_index("s") * 2 + lax.axis_index("c")


def _routing_body(x_ref, gw_ref, e_ref, p_ref, xb_ref):
    x = x_ref[...]
    logits = lax.dot_general(
        x, gw_ref[...], (((1,), (1,)), ((), ())),
        preferred_element_type=jnp.float32,
    )
    lane = lax.broadcasted_iota(jnp.int32, (T, EPAD), 1)
    logits = jnp.where(lane < N_EXP, logits, jnp.float32(-1e30))
    m = jnp.max(logits, axis=1, keepdims=True)
    ex = jnp.where(lane < N_EXP, jnp.exp(logits - m), 0.0)
    probs = ex / jnp.sum(ex, axis=1, keepdims=True)
    p0 = jnp.max(probs, axis=1, keepdims=True)
    i0 = jnp.min(jnp.where(probs == p0, lane, EPAD), axis=1, keepdims=True)
    probs2 = jnp.where(lane == i0, -1.0, probs)
    p1 = jnp.max(probs2, axis=1, keepdims=True)
    i1 = jnp.min(jnp.where(probs2 == p1, lane, EPAD), axis=1, keepdims=True)
    denom = p0 + p1
    e_ref[...] = jnp.concatenate([i0, i1], axis=1)
    p_ref[...] = jnp.concatenate([p0 / denom, p1 / denom], axis=1)
    xb_ref[...] = x.astype(jnp.bfloat16)


def _routing(x2d, gate_w):
    gwp = jnp.zeros((EPAD, D_MODEL), jnp.float32).at[:N_EXP].set(gate_w)
    return pl.pallas_call(
        _routing_body,
        out_shape=(
            jax.ShapeDtypeStruct((T, 2), jnp.int32),
            jax.ShapeDtypeStruct((T, 2), jnp.float32),
            jax.ShapeDtypeStruct((T, D_MODEL), jnp.bfloat16),
        ),
    )(x2d, gwp)


def _dispatch_body(e_hbm, p_hbm, tok_hbm, dest_hbm, eob_hbm,
                   a_v, p_v, tok_v, dest_v, eb_v, r_s):
    @pl.when(_wid() == 0)
    def _():
        pltpu.sync_copy(e_hbm, a_v)
        pltpu.sync_copy(p_hbm, p_v)
        lanes = lax.iota(jnp.int32, L)

        def cnt_body(c, acc):
            av = a_v[pl.ds(c * L, L)]
            for e in range(N_EXP):
                pref = plsc.cumsum((av == e).astype(jnp.int32))
                acc = acc + jnp.where(lanes == e, jnp.max(pref), 0)
            return acc
        counts = lax.fori_loop(0, A // L, cnt_body, jnp.zeros((L,), jnp.int32))

        pc = ((counts + (BM - 1)) >> 9) << 9
        pstart_incl = plsc.cumsum(pc)
        pstart = pstart_incl - pc

        pos = lanes * BM
        eobv = jnp.zeros((L,), jnp.int32)
        for e in range(N_EXP):
            se = jnp.max(jnp.where(lanes == e, pstart_incl, jnp.int32(-2147483647)))
            eobv = eobv + jnp.where(pos >= se, 1, 0)
        nreal = jnp.max(
            jnp.where(lanes == N_EXP - 1, pstart_incl, jnp.int32(-2147483647))
        ) >> 9
        eb_v[...] = jnp.where(
            lanes == L - 1, nreal, jnp.minimum(eobv, N_EXP - 1)
        )

        for e in range(N_EXP):
            r_s[e] = jnp.max(jnp.where(lanes == e, pstart, jnp.int32(-2147483647)))

        def sort_body(c, _):
            base = c * L
            av = a_v[pl.ds(base, L)]
            pv = p_v[pl.ds(base, L)]
            toks = (base + lanes) >> 1
            dest = jnp.zeros((L,), jnp.int32)
            for e in range(N_EXP):
                m = av == e
                pref = plsc.cumsum(m.astype(jnp.int32))
                re = r_s[e]
                dest = jnp.where(m, re + pref - 1, dest)
                r_s[e] = re + jnp.max(pref)
            dest_v[pl.ds(base, L)] = dest
            plsc.store_scatter(tok_v, [dest], toks)
            return 0
        lax.fori_loop(0, A // L, sort_body, 0)

        for b in range(NB):
            pltpu.sync_copy(
                tok_v.at[pl.ds(b * BM, BM)], tok_hbm.at[b, 0]
            )
        pltpu.sync_copy(dest_v, dest_hbm)
        pltpu.sync_copy(eb_v, eob_hbm)


def _dispatch(e01, p01):
    return pl.kernel(
        _dispatch_body,
        out_type=(
            jax.ShapeDtypeStruct((NB, 1, BM), jnp.int32),
            jax.ShapeDtypeStruct((A,), jnp.int32),
            jax.ShapeDtypeStruct((L,), jnp.int32),
        ),
        mesh=_sc_mesh(),
        scratch_types=[
            pltpu.VMEM((A,), jnp.int32),
            pltpu.VMEM((A,), jnp.float32),
            pltpu.VMEM((PMAX,), jnp.int32),
            pltpu.VMEM((A,), jnp.int32),
            pltpu.VMEM((L,), jnp.int32),
            pltpu.SMEM((L,), jnp.int32),
        ],
        compiler_params=pltpu.CompilerParams(needs_layout_passes=False),
    )(e01, p01)


FFC = 1408
NF = D_FF // FFC


def _moe_body(eob_s, x_ref, tok_ref, w1_ref, w3_ref, w2_ref, o_ref,
              xb_s):
    i = pl.program_id(0)
    f = pl.program_id(1)

    @pl.when(i < eob_s[L - 1])
    def _():
        @pl.when(f == 0)
        def _():
            sel = (
                lax.broadcasted_iota(jnp.int32, (T, BM), 0) == tok_ref[0]
            ).astype(jnp.bfloat16)
            xb_s[...] = lax.dot_general(
                sel, x_ref[...], (((0,), (0,)), ((), ())),
                preferred_element_type=jnp.float32,
            ).astype(jnp.bfloat16)

        xb = xb_s[...]
        g = lax.dot_general(
            xb, w1_ref[0].astype(jnp.bfloat16), (((1,), (1,)), ((), ())),
            preferred_element_type=jnp.float32,
        )
        v = lax.dot_general(
            xb, w3_ref[0].astype(jnp.bfloat16), (((1,), (1,)), ((), ())),
            preferred_element_type=jnp.float32,
        )
        h = (g * lax.logistic(g) * v).astype(jnp.bfloat16)
        o = lax.dot_general(
            h, w2_ref[0].astype(jnp.bfloat16), (((1,), (1,)), ((), ())),
            preferred_element_type=jnp.float32,
        )
        @pl.when(f == 0)
        def _():
            o_ref[...] = o

        @pl.when(f != 0)
        def _():
            o_ref[...] += o


def _expert_mm(eob, xb16, tok3, w1, w3, w2):
    grid_spec = pltpu.PrefetchScalarGridSpec(
        num_scalar_prefetch=1,
        grid=(NB, NF),
        in_specs=[
            pl.BlockSpec((T, D_MODEL), lambda i, f, eob: (0, 0)),
            pl.BlockSpec((1, 1, BM), lambda i, f, eob: (i, 0, 0)),
            pl.BlockSpec((1, FFC, D_MODEL), lambda i, f, eob: (eob[i], f, 0)),
            pl.BlockSpec((1, FFC, D_MODEL), lambda i, f, eob: (eob[i], f, 0)),
            pl.BlockSpec((1, D_MODEL, FFC), lambda i, f, eob: (eob[i], 0, f)),
        ],
        out_specs=pl.BlockSpec((BM, D_MODEL), lambda i, f, eob: (i, 0)),
        scratch_shapes=[pltpu.VMEM((BM, D_MODEL), jnp.bfloat16)],
    )
    return pl.pallas_call(
        _moe_body,
        grid_spec=grid_spec,
        out_shape=jax.ShapeDtypeStruct((PMAX, D_MODEL), jnp.float32),
        compiler_params=pltpu.CompilerParams(
            dimension_semantics=("arbitrary", "arbitrary"),
            vmem_limit_bytes=100 * 1024 * 1024,
        ),
    )(eob, xb16, tok3, w1, w3, w2)


def _combine_body(o_hbm, dest_hbm, p_hbm, out_hbm, idx_v, p_v, rows_v,
                  out_v, sem):
    wid = _wid()
    abase = wid * (A // NW)
    pltpu.sync_copy(dest_hbm.at[pl.ds(abase, A // NW)], idx_v)
    pltpu.sync_copy(p_hbm.at[pl.ds(abase, A // NW)], p_v)
    for half in range(2):
        pltpu.async_copy(
            o_hbm.at[idx_v.at[pl.ds(half * 64, 64)]], rows_v, sem
        ).wait()

        def add_body(j, _):
            jj = half * 64 + 2 * j
            pa = plsc.load_gather(p_v, [jnp.full((L,), jj, jnp.int32)])
            pb = plsc.load_gather(p_v, [jnp.full((L,), jj + 1, jnp.int32)])
            for c in range(D_MODEL // L):
                sl = pl.ds(c * L, L)
                out_v[j, sl] = (
                    rows_v[2 * j, sl] * pa + rows_v[2 * j + 1, sl] * pb
                )
            return 0
        lax.fori_loop(0, 32, add_body, 0)
        pltpu.sync_copy(
            out_v, out_hbm.at[pl.ds(wid * 64 + half * 32, 32)]
        )


def _combine(osrt, dest, p01):
    return pl.kernel(
        _combine_body,
        out_type=jax.ShapeDtypeStruct((T, D_MODEL), jnp.float32),
        mesh=_sc_mesh(),
        scratch_types=[
            pltpu.VMEM((A // NW,), jnp.int32),
            pltpu.VMEM((A // NW,), jnp.float32),
            pltpu.VMEM((64, D_MODEL), jnp.float32),
            pltpu.VMEM((32, D_MODEL), jnp.float32),
            pltpu.SemaphoreType.DMA,
        ],
        compiler_params=pltpu.CompilerParams(needs_layout_passes=False),
    )(osrt, dest, p01)


@jax.jit
def kernel(x, gate_w, w1, w2, w3):
    B, S, D = x.shape
    x2d = x.reshape(-1, D)
    e2, p2, xb16 = _routing(x2d, gate_w)
    p01 = p2.reshape(-1)
    tok3, dest, eob = _dispatch(e2.reshape(-1), p01)
    osrt = _expert_mm(eob, xb16, tok3, w1, w3, w2)
    out2d = _combine(osrt, dest, p01)
    return out2d.reshape(B, S, D)


# speedup vs baseline: 1.0141x; 1.0141x over previous
import functools

import jax
import jax.numpy as jnp
from jax import lax
from jax.experimental import pallas as pl
from jax.experimental.pallas import tpu as pltpu
from jax.experimental.pallas import tpu_sc as plsc

D_MODEL = 1024
D_FF = 2816
N_EXP = 8
EPAD = 128
T = 2048
A = 4096
BM = 512
NB = 15
PMAX = NB * BM
NW = 32
L = 16

_sc_mesh = functools.partial(
    plsc.VectorSubcoreMesh, core_axis_name="c", subcore_axis_name="s"
)


def _wid():
    return lax.axis_index("s") * 2 + lax.axis_index("c")


def _routing_body(x_ref, gw_ref, e_ref, p_ref, xb_ref):
    x = x_ref[...]
    logits = lax.dot_general(
        x, gw_ref[...], (((1,), (1,)), ((), ())),
        preferred_element_type=jnp.float32,
    )
    lane = lax.broadcasted_iota(jnp.int32, (T, EPAD), 1)
    logits = jnp.where(lane < N_EXP, logits, jnp.float32(-1e30))
    m = jnp.max(logits, axis=1, keepdims=True)
    ex = jnp.where(lane < N_EXP, jnp.exp(logits - m), 0.0)
    probs = ex / jnp.sum(ex, axis=1, keepdims=True)
    p0 = jnp.max(probs, axis=1, keepdims=True)
    i0 = jnp.min(jnp.where(probs == p0, lane, EPAD), axis=1, keepdims=True)
    probs2 = jnp.where(lane == i0, -1.0, probs)
    p1 = jnp.max(probs2, axis=1, keepdims=True)
    i1 = jnp.min(jnp.where(probs2 == p1, lane, EPAD), axis=1, keepdims=True)
    denom = p0 + p1
    e_ref[...] = jnp.concatenate([i0, i1], axis=1)
    p_ref[...] = jnp.concatenate([p0 / denom, p1 / denom], axis=1)
    xb_ref[...] = x.astype(jnp.bfloat16)


def _routing(x2d, gate_w):
    gwp = jnp.zeros((EPAD, D_MODEL), jnp.float32).at[:N_EXP].set(gate_w)
    return pl.pallas_call(
        _routing_body,
        out_shape=(
            jax.ShapeDtypeStruct((T, 2), jnp.int32),
            jax.ShapeDtypeStruct((T, 2), jnp.float32),
            jax.ShapeDtypeStruct((T, D_MODEL), jnp.bfloat16),
        ),
    )(x2d, gwp)


def _dispatch_body(e_hbm, p_hbm, tok_hbm, w_hbm, dest_hbm, eob_hbm,
                   a_v, p_v, tok_v, w_v, dest_v, eb_v, r_s):
    @pl.when(_wid() == 0)
    def _():
        pltpu.sync_copy(e_hbm, a_v)
        pltpu.sync_copy(p_hbm, p_v)
        lanes = lax.iota(jnp.int32, L)

        def cnt_body(c, acc):
            av = a_v[pl.ds(c * L, L)]
            for e in range(N_EXP):
                pref = plsc.cumsum((av == e).astype(jnp.int32))
                acc = acc + jnp.where(lanes == e, jnp.max(pref), 0)
            return acc
        counts = lax.fori_loop(0, A // L, cnt_body, jnp.zeros((L,), jnp.int32))

        pc = ((counts + (BM - 1)) >> 9) << 9
        pstart_incl = plsc.cumsum(pc)
        pstart = pstart_incl - pc

        pos = lanes * BM
        eobv = jnp.zeros((L,), jnp.int32)
        for e in range(N_EXP):
            se = jnp.max(jnp.where(lanes == e, pstart_incl, jnp.int32(-2147483647)))
            eobv = eobv + jnp.where(pos >= se, 1, 0)
        nreal = jnp.max(
            jnp.where(lanes == N_EXP - 1, pstart_incl, jnp.int32(-2147483647))
        ) >> 9
        eb_v[...] = jnp.where(
            lanes == L - 1, nreal, jnp.minimum(eobv, N_EXP - 1)
        )

        zi = jnp.zeros((L,), jnp.int32)
        zf = jnp.zeros((L,), jnp.float32)

        def z_body(i, _):
            tok_v[pl.ds(i * L, L)] = zi
            w_v[pl.ds(i * L, L)] = zf
            return 0
        lax.fori_loop(0, PMAX // L, z_body, 0)

        for e in range(N_EXP):
            r_s[e] = jnp.max(jnp.where(lanes == e, pstart, jnp.int32(-2147483647)))

        def sort_body(c, _):
            base = c * L
            av = a_v[pl.ds(base, L)]
            pv = p_v[pl.ds(base, L)]
            toks = (base + lanes) >> 1
            dest = jnp.zeros((L,), jnp.int32)
            for e in range(N_EXP):
                m = av == e
                pref = plsc.cumsum(m.astype(jnp.int32))
                re = r_s[e]
                dest = jnp.where(m, re + pref - 1, dest)
                r_s[e] = re + jnp.max(pref)
            dest_v[pl.ds(base, L)] = dest
            plsc.store_scatter(tok_v, [dest], toks)
            plsc.store_scatter(w_v, [dest], pv)
            return 0
        lax.fori_loop(0, A // L, sort_body, 0)

        pltpu.sync_copy(tok_v, tok_hbm)
        pltpu.sync_copy(w_v, w_hbm)
        pltpu.sync_copy(dest_v, dest_hbm)
        pltpu.sync_copy(eb_v, eob_hbm)


def _dispatch(e01, p01):
    return pl.kernel(
        _dispatch_body,
        out_type=(
            jax.ShapeDtypeStruct((PMAX,), jnp.int32),
            jax.ShapeDtypeStruct((PMAX,), jnp.float32),
            jax.ShapeDtypeStruct((A,), jnp.int32),
            jax.ShapeDtypeStruct((L,), jnp.int32),
        ),
        mesh=_sc_mesh(),
        scratch_types=[
            pltpu.VMEM((A,), jnp.int32),
            pltpu.VMEM((A,), jnp.float32),
            pltpu.VMEM((PMAX,), jnp.int32),
            pltpu.VMEM((PMAX,), jnp.float32),
            pltpu.VMEM((A,), jnp.int32),
            pltpu.VMEM((L,), jnp.int32),
            pltpu.SMEM((L,), jnp.int32),
        ],
        compiler_params=pltpu.CompilerParams(needs_layout_passes=False),
    )(e01, p01)


FFC = 1408
NF = D_FF // FFC


def _moe_body(eob_s, x_ref, tok_ref, w1_ref, w3_ref, w2_ref, wc_ref, o_ref,
              xb_s):
    i = pl.program_id(0)
    f = pl.program_id(1)

    @pl.when(i < eob_s[L - 1])
    def _():
        @pl.when(f == 0)
        def _():
            sel = (
                lax.broadcasted_iota(jnp.int32, (BM, T), 1) == tok_ref[...]
            ).astype(jnp.bfloat16)
            xb_s[...] = lax.dot_general(
                sel, x_ref[...], (((1,), (0,)), ((), ())),
                preferred_element_type=jnp.float32,
            ).astype(jnp.bfloat16)

        xb = xb_s[...]
        g = lax.dot_general(
            xb, w1_ref[0].astype(jnp.bfloat16), (((1,), (1,)), ((), ())),
            preferred_element_type=jnp.float32,
        )
        v = lax.dot_general(
            xb, w3_ref[0].astype(jnp.bfloat16), (((1,), (1,)), ((), ())),
            preferred_element_type=jnp.float32,
        )
        h = (g * lax.logistic(g) * v).astype(jnp.bfloat16)
        o = lax.dot_general(
            h, w2_ref[0].astype(jnp.bfloat16), (((1,), (1,)), ((), ())),
            preferred_element_type=jnp.float32,
        )
        contrib = o * wc_ref[...]

        @pl.when(f == 0)
        def _():
            o_ref[...] = contrib

        @pl.when(f != 0)
        def _():
            o_ref[...] += contrib


def _expert_mm(eob, xb16, tok2, w1, w3, w2, wcol):
    grid_spec = pltpu.PrefetchScalarGridSpec(
        num_scalar_prefetch=1,
        grid=(NB, NF),
        in_specs=[
            pl.BlockSpec((T, D_MODEL), lambda i, f, eob: (0, 0)),
            pl.BlockSpec((BM, 1), lambda i, f, eob: (i, 0)),
            pl.BlockSpec((1, FFC, D_MODEL), lambda i, f, eob: (eob[i], f, 0)),
            pl.BlockSpec((1, FFC, D_MODEL), lambda i, f, eob: (eob[i], f, 0)),
            pl.BlockSpec((1, D_MODEL, FFC), lambda i, f, eob: (eob[i], 0, f)),
            pl.BlockSpec((BM, 1), lambda i, f, eob: (i, 0)),
        ],
        out_specs=pl.BlockSpec((BM, D_MODEL), lambda i, f, eob: (i, 0)),
        scratch_shapes=[pltpu.VMEM((BM, D_MODEL), jnp.bfloat16)],
    )
    return pl.pallas_call(
        _moe_body,
        grid_spec=grid_spec,
        out_shape=jax.ShapeDtypeStruct((PMAX, D_MODEL), jnp.float32),
        compiler_params=pltpu.CompilerParams(
            dimension_semantics=("arbitrary", "arbitrary"),
            vmem_limit_bytes=100 * 1024 * 1024,
        ),
    )(eob, xb16, tok2, w1, w3, w2, wcol)


def _combine_body(o_hbm, dest_hbm, out_hbm, idx_v, rows_v, out_v, sem):
    wid = _wid()
    abase = wid * (A // NW)
    pltpu.sync_copy(dest_hbm.at[pl.ds(abase, A // NW)], idx_v)
    for half in range(2):
        pltpu.async_copy(
            o_hbm.at[idx_v.at[pl.ds(half * 64, 64)]], rows_v, sem
        ).wait()

        def add_body(c, _):
            sl = pl.ds(c * L, L)
            for j in range(32):
                out_v[j, sl] = rows_v[2 * j, sl] + rows_v[2 * j + 1, sl]
            return 0
        lax.fori_loop(0, D_MODEL // L, add_body, 0)
        pltpu.sync_copy(
            out_v, out_hbm.at[pl.ds(wid * 64 + half * 32, 32)]
        )


def _combine(osrt, dest):
    return pl.kernel(
        _combine_body,
        out_type=jax.ShapeDtypeStruct((T, D_MODEL), jnp.float32),
        mesh=_sc_mesh(),
        scratch_types=[
            pltpu.VMEM((A // NW,), jnp.int32),
            pltpu.VMEM((64, D_MODEL), jnp.float32),
            pltpu.VMEM((32, D_MODEL), jnp.float32),
            pltpu.SemaphoreType.DMA,
        ],
        compiler_params=pltpu.CompilerParams(needs_layout_passes=False),
    )(osrt, dest)


@jax.jit
def kernel(x, gate_w, w1, w2, w3):
    B, S, D = x.shape
    x2d = x.reshape(-1, D)
    e2, p2, xb16 = _routing(x2d, gate_w)
    tok, wsrt, dest, eob = _dispatch(e2.reshape(-1), p2.reshape(-1))
    osrt = _expert_mm(
        eob, xb16, tok.reshape(-1, 1), w1, w3, w2, wsrt.reshape(-1, 1),
    )
    out2d = _combine(osrt, dest)
    return out2d.reshape(B, S, D)
